# baseline (speedup 1.0000x reference)
import functools

import jax
import jax.numpy as jnp
from jax.experimental import pallas as pl
from jax.experimental.pallas import tpu as pltpu

_GRID_R, _GRID_C = 28, 28
_BOX_NUM = 3
_CLASS_NUM = 80
_F = 5 * _BOX_NUM + _CLASS_NUM
_B = 64
_N = _B * _GRID_R * _GRID_C
_STEPS = 7
_BLK_N = _N // _STEPS
_M = _N // 8
_BLK_L = _M // _STEPS


def _body(p_ref, gt_ref, pT_ref, gtT_ref, ii_ref, jj_ref, out_ref, acc_ref):
    g = pl.program_id(0)

    @pl.when(g == 0)
    def _init():
        for k in range(6):
            acc_ref[k] = 0.0

    pos_n = (gt_ref[:, 0:1] > 0.0).astype(jnp.float32)
    d = p_ref[:, 15:_F] - gt_ref[:, 15:_F]
    class_part = jnp.sum(d * d * pos_n)

    ii = ii_ref[...]
    jj = jj_ref[...]
    pos = gtT_ref[0] > 0.0

    c0 = pT_ref[0]
    c1 = pT_ref[5]
    c2 = pT_ref[10]
    best = (
        (c0 >= c1) & (c0 >= c2),
        (c1 > c0) & (c1 >= c2),
        (c2 > c0) & (c2 > c1),
    )

    zero = jnp.zeros_like(c0)
    xy_p = zero
    wh_p = zero
    pc_p = zero
    nc_p = zero
    for k in range(_BOX_NUM):
        ck = (c0, c1, c2)[k]
        m = (pos & best[k]).astype(jnp.float32)
        px = pT_ref[5 * k + 1]
        py = pT_ref[5 * k + 2]
        pw = pT_ref[5 * k + 3]
        ph = pT_ref[5 * k + 4]
        gx = gtT_ref[5 * k + 1]
        gy = gtT_ref[5 * k + 2]
        gw = gtT_ref[5 * k + 3]
        gh = gtT_ref[5 * k + 4]

        gxc = gx * float(_GRID_C) - jj
        gyc = gy * float(_GRID_R) - ii
        dx = px - gxc
        dy = py - gyc
        dw = pw - gw
        dh = ph - gh
        xy_p = xy_p + m * (dx * dx + dy * dy)
        wh_p = wh_p + m * (dw * dw + dh * dh)

        pxg = (px + jj) / float(_GRID_C)
        pyg = (py + ii) / float(_GRID_R)
        ax1 = pxg - pw * 0.5
        ax2 = pxg + pw * 0.5
        ay1 = pyg - ph * 0.5
        ay2 = pyg + ph * 0.5
        bx1 = gx - gw * 0.5
        bx2 = gx + gw * 0.5
        by1 = gy - gh * 0.5
        by2 = gy + gh * 0.5
        iw = jnp.maximum(jnp.minimum(ax2, bx2) - jnp.maximum(ax1, bx1), 0.0)
        ih = jnp.maximum(jnp.minimum(ay2, by2) - jnp.maximum(ay1, by1), 0.0)
        inter = iw * ih
        area_a = jnp.maximum(pw, 0.0) * jnp.maximum(ph, 0.0)
        area_b = jnp.maximum(gw, 0.0) * jnp.maximum(gh, 0.0)
        iou = inter / (area_a + area_b - inter + 1e-10)

        dc = ck - iou
        pc_p = pc_p + m * dc * dc
        nc_p = nc_p + (1.0 - m) * ck * ck

    acc_ref[0] = acc_ref[0] + class_part
    acc_ref[1] = acc_ref[1] + jnp.sum(xy_p)
    acc_ref[2] = acc_ref[2] + jnp.sum(wh_p)
    acc_ref[3] = acc_ref[3] + jnp.sum(pc_p)
    acc_ref[4] = acc_ref[4] + jnp.sum(nc_p)
    acc_ref[5] = acc_ref[5] + jnp.sum(pos.astype(jnp.float32))

    @pl.when(g == _STEPS - 1)
    def _finish():
        npos = acc_ref[5]
        class_loss = acc_ref[0] / jnp.maximum(float(_CLASS_NUM) * npos, 1.0)
        xy_loss = acc_ref[1] / jnp.maximum(2.0 * npos, 1.0)
        wh_loss = acc_ref[2] / jnp.maximum(2.0 * npos, 1.0)
        pos_conf = acc_ref[3] / jnp.maximum(npos, 1.0)
        neg_conf = acc_ref[4] / jnp.maximum(float(_BOX_NUM * _N) - npos, 1.0)
        out_ref[0] = (class_loss + 2.0 * pos_conf + 0.5 * neg_conf
                      + 5.0 * xy_loss + 5.0 * wh_loss)
        out_ref[1] = class_loss
        out_ref[2] = xy_loss
        out_ref[3] = wh_loss
        out_ref[4] = pos_conf
        out_ref[5] = neg_conf


@functools.partial(jax.jit, static_argnames=("interpret",))
def _yolo_loss(p, gt, interpret=False):
    p2 = p.reshape(_N, _F)
    gt2 = gt.reshape(_N, _F)
    pT = jnp.transpose(p2[:, :15]).reshape(15, 8, _M)
    gtT = jnp.transpose(gt2[:, :15]).reshape(15, 8, _M)
    n = jnp.arange(_N, dtype=jnp.int32)
    ii = ((n // _GRID_C) % _GRID_R).astype(jnp.float32).reshape(8, _M)
    jj = (n % _GRID_C).astype(jnp.float32).reshape(8, _M)

    out = pl.pallas_call(
        _body,
        grid=(_STEPS,),
        in_specs=[
            pl.BlockSpec((_BLK_N, _F), lambda g: (g, 0)),
            pl.BlockSpec((_BLK_N, _F), lambda g: (g, 0)),
            pl.BlockSpec((15, 8, _BLK_L), lambda g: (0, 0, g)),
            pl.BlockSpec((15, 8, _BLK_L), lambda g: (0, 0, g)),
            pl.BlockSpec((8, _BLK_L), lambda g: (0, g)),
            pl.BlockSpec((8, _BLK_L), lambda g: (0, g)),
        ],
        out_specs=pl.BlockSpec(memory_space=pltpu.SMEM),
        out_shape=jax.ShapeDtypeStruct((6,), jnp.float32),
        scratch_shapes=[pltpu.SMEM((6,), jnp.float32)],
        interpret=interpret,
    )(p2, gt2, pT, gtT, ii, jj)
    return (out[0], out[1], out[2], out[3], out[4], out[5])


def kernel(p, gt):
    return _yolo_loss(p, gt)
